# Initial kernel scaffold; baseline (speedup 1.0000x reference)
import jax
import jax.numpy as jnp
from jax.experimental import pallas as pl


def kernel(hidden_states, router_params, expert_params):
    raise NotImplementedError("write your pallas kernel here")


# speedup vs baseline: 2.3156x; 2.3156x over previous
import functools

import jax
import jax.numpy as jnp
from jax.experimental import pallas as pl
from jax.experimental.pallas import tpu as pltpu

_H = 1024
_INTER = 2048
_NE = 12
_NH = 8
_HD = _H // _NH
_NSPEC = 12
_LANE = 128
_TB = 256


def _expert_shapes(e):
    if e in (3, 4):
        return [(_H, _INTER * 2), (_INTER * 2, _INTER), (_INTER, _H)]
    if e == 5:
        return [(_H, _INTER), (_INTER, _INTER), (_INTER, _INTER // 2),
                (_INTER // 2, _H)]
    return [(_H, _INTER), (_INTER, _INTER // 2), (_INTER // 2, _H)]


def _gelu_exact(x):
    return 0.5 * x * (1.0 + jax.lax.erf(x * 0.7071067811865476))


def _pad_lane(a, n=_LANE):
    return jnp.pad(a, ((0, 0), (0, n - a.shape[-1])))


def _masked_softmax_vec(logits, ids, nvalid):
    valid = ids < nvalid
    el = jnp.where(valid, logits, -jnp.inf)
    m = jnp.max(el)
    ex = jnp.where(valid, jnp.exp(el - m), 0.0)
    return ex / jnp.sum(ex)


def _router_body(x_ref, wq_ref, wk_ref, wv_ref, bq_ref, bk_ref, bv_ref,
                 wo_ref, bo_ref, wc1_ref, bc1_ref, wc2_ref, bc2_ref,
                 wr1a_ref, wr1b_ref, br1_ref, wr2_ref, br2_ref,
                 wu1_ref, bu1_ref, wu2_ref, bu2_ref,
                 spec_ref, urg_ref, w_ref, idx_ref, acc_ref):
    h = pl.program_id(0)
    hp = jax.lax.Precision.HIGHEST

    @pl.when(h == 0)
    def _init():
        acc_ref[...] = jnp.zeros_like(acc_ref)

    x = x_ref[...]
    S = x.shape[0]
    q = jnp.dot(x, wq_ref[0], precision=hp) + bq_ref[0]
    k = jnp.dot(x, wk_ref[0], precision=hp) + bk_ref[0]
    v = jnp.dot(x, wv_ref[0], precision=hp) + bv_ref[0]
    scale = 1.0 / jnp.sqrt(float(_HD))
    QB = 512

    colsum = jnp.zeros((1, S), jnp.float32)
    for i in range(S // QB):
        qc = q[i * QB:(i + 1) * QB]
        s = jax.lax.dot_general(qc, k, (((1,), (1,)), ((), ())),
                                precision=hp) * scale
        s = s - jnp.max(s, axis=-1, keepdims=True)
        es = jnp.exp(s)
        p = es / jnp.sum(es, axis=-1, keepdims=True)
        colsum = colsum + jnp.sum(p, axis=0, keepdims=True)
    am = colsum * (1.0 / S)
    mh = jnp.dot(am, v, precision=hp)
    acc_ref[...] += jnp.dot(mh, wo_ref[0], precision=hp)

    @pl.when(h == _NH - 1)
    def _tail():
        ids = jax.lax.broadcasted_iota(jnp.int32, (1, _LANE), 1)
        pooled = acc_ref[...] + bo_ref[...]
        c = jnp.maximum(jnp.dot(pooled, wc1_ref[...], precision=hp)
                        + bc1_ref[...], 0.0)
        spec_logits = jnp.dot(c, wc2_ref[...], precision=hp) + bc2_ref[...]
        spec_probs = _masked_softmax_vec(spec_logits, ids, _NSPEC)
        spec_ref[...] = spec_probs
        r = jnp.maximum(jnp.dot(pooled, wr1a_ref[...], precision=hp)
                        + jnp.dot(spec_probs, wr1b_ref[...], precision=hp)
                        + br1_ref[...], 0.0)
        exp_logits = jnp.dot(r, wr2_ref[...], precision=hp) + br2_ref[...]
        u = jnp.maximum(jnp.dot(pooled, wu1_ref[...], precision=hp)
                        + bu1_ref[...], 0.0)
        urg = jnp.dot(u, wu2_ref[...], precision=hp) + bu2_ref[...]
        urg_ref[...] = jax.nn.sigmoid(urg)
        probs = _masked_softmax_vec(exp_logits, ids, _NE)
        wv, iv = [], []
        pcur = probs
        for _ in range(3):
            mj = jnp.max(pcur)
            ij = jnp.min(jnp.where(pcur == mj, ids, _NE + _LANE))
            wv.append(mj)
            iv.append(ij)
            pcur = jnp.where(ids == ij, -1.0, pcur)
        e1 = jnp.exp(wv[1] - wv[0])
        e2 = jnp.exp(wv[2] - wv[0])
        z = 1.0 + e1 + e2
        wn = [1.0 / z, e1 / z, e2 / z]
        w_vec = jnp.where(ids == 0, wn[0],
                          jnp.where(ids == 1, wn[1],
                                    jnp.where(ids == 2, wn[2], 0.0)))
        i_vec = jnp.where(ids == 0, iv[0],
                          jnp.where(ids == 1, iv[1],
                                    jnp.where(ids == 2, iv[2], 0)))
        w_ref[...] = w_vec
        idx_ref[...] = i_vec.astype(jnp.int32)


def _run_router(x, rp, S):
    wqkv = rp['Wqkv'].reshape(_H, 3, _NH, _HD)
    wq = wqkv[:, 0].transpose(1, 0, 2)
    wk = wqkv[:, 1].transpose(1, 0, 2)
    wv = wqkv[:, 2].transpose(1, 0, 2)
    bqkv = rp['bqkv'].reshape(3, _NH, 1, _HD)
    bq, bk, bv = bqkv[0], bqkv[1], bqkv[2]
    wo = rp['Wo'].reshape(_NH, _HD, _H)
    bo = rp['bo'][None, :]
    wc1 = rp['Wc1']
    bc1 = rp['bc1'][None, :]
    wc2 = _pad_lane(rp['Wc2'])
    bc2 = _pad_lane(rp['bc2'][None, :])
    wr1a = rp['Wr1'][:_H]
    wr1b = jnp.pad(rp['Wr1'][_H:], ((0, _LANE - _NSPEC), (0, 0)))
    br1 = rp['br1'][None, :]
    wr2 = _pad_lane(rp['Wr2'])
    br2 = _pad_lane(rp['br2'][None, :])
    wu1 = rp['Wu1']
    bu1 = rp['bu1'][None, :]
    wu2 = _pad_lane(rp['Wu2'])
    bu2 = _pad_lane(rp['bu2'][None, :])

    def head_spec(shape):
        nd = len(shape)
        return pl.BlockSpec((1,) + shape[1:],
                            lambda i: (i,) + (0,) * (nd - 1))

    def full_spec(shape):
        nd = len(shape)
        return pl.BlockSpec(shape, lambda i: (0,) * nd)

    args = (x, wq, wk, wv, bq, bk, bv, wo, bo, wc1, bc1, wc2, bc2,
            wr1a, wr1b, br1, wr2, br2, wu1, bu1, wu2, bu2)
    in_specs = [full_spec(x.shape)]
    for a in (wq, wk, wv, bq, bk, bv, wo):
        in_specs.append(head_spec(a.shape))
    for a in args[8:]:
        in_specs.append(full_spec(a.shape))

    out_shape = [
        jax.ShapeDtypeStruct((1, _LANE), jnp.float32),
        jax.ShapeDtypeStruct((1, _LANE), jnp.float32),
        jax.ShapeDtypeStruct((1, _LANE), jnp.float32),
        jax.ShapeDtypeStruct((1, _LANE), jnp.int32),
    ]
    out_specs = [full_spec((1, _LANE))] * 4
    return pl.pallas_call(
        _router_body,
        grid=(_NH,),
        in_specs=in_specs,
        out_specs=out_specs,
        out_shape=out_shape,
        scratch_shapes=[pltpu.VMEM((1, _H), jnp.float32)],
    )(*args)


def _expert_body(n, *refs):
    x_ref = refs[0]
    wrefs = refs[1:1 + n]
    brefs = refs[1 + n:1 + 2 * n]
    (lng_ref, lnb_ref, wc1_ref, bc1_ref, wc2_ref, bc2_ref,
     eo_ref, conf_ref) = refs[1 + 2 * n:]
    h = x_ref[...]
    for i in range(n):
        a = jnp.dot(h, wrefs[i][...],
                    preferred_element_type=jnp.float32) + brefs[i][...]
        if i < n - 1:
            h = _gelu_exact(a).astype(jnp.bfloat16)
        else:
            h = a
    mu = jnp.mean(h, axis=-1, keepdims=True)
    var = jnp.mean((h - mu) ** 2, axis=-1, keepdims=True)
    out = (h - mu) / jnp.sqrt(var + 1e-5) * lng_ref[...] + lnb_ref[...]
    eo_ref[...] = out
    ob = out.astype(jnp.bfloat16)
    c = jnp.maximum(jnp.dot(ob, wc1_ref[...],
                            preferred_element_type=jnp.float32)
                    + bc1_ref[...], 0.0)
    conf_ref[...] = jax.nn.sigmoid(
        jnp.dot(c.astype(jnp.bfloat16), wc2_ref[...],
                preferred_element_type=jnp.float32) + bc2_ref[...])


def _run_expert(p, n, xb, S):
    ws = [w.astype(jnp.bfloat16) for w in p['Ws']]
    bs = [b[None, :] for b in p['bs']]
    lng = p['ln_g'][None, :]
    lnb = p['ln_b'][None, :]
    wc1 = p['Wc1'].astype(jnp.bfloat16)
    bc1 = p['bc1'][None, :]
    wc2 = _pad_lane(p['Wc2']).astype(jnp.bfloat16)
    bc2 = _pad_lane(p['bc2'][None, :])
    args = (xb, *ws, *bs, lng, lnb, wc1, bc1, wc2, bc2)

    def full_spec(shape):
        nd = len(shape)
        return pl.BlockSpec(shape, lambda i: (0,) * nd)

    in_specs = [pl.BlockSpec((_TB, _H), lambda i: (i, 0))]
    in_specs += [full_spec(a.shape) for a in args[1:]]
    out_shape = [
        jax.ShapeDtypeStruct((S, _H), jnp.float32),
        jax.ShapeDtypeStruct((S, _LANE), jnp.float32),
    ]
    out_specs = [pl.BlockSpec((_TB, _H), lambda i: (i, 0)),
                 pl.BlockSpec((_TB, _LANE), lambda i: (i, 0))]
    return pl.pallas_call(
        functools.partial(_expert_body, n),
        grid=(S // _TB,),
        in_specs=in_specs,
        out_specs=out_specs,
        out_shape=out_shape,
    )(*args)


def _combine_body(w_ref, eo0_ref, eo1_ref, eo2_ref, c0_ref, c1_ref, c2_ref,
                  out_ref, tc_ref):
    c0 = c0_ref[...]
    c1 = c1_ref[...]
    c2 = c2_ref[...]
    out_ref[...] = ((w_ref[0] * c0[:, :1]) * eo0_ref[...]
                    + (w_ref[1] * c1[:, :1]) * eo1_ref[...]
                    + (w_ref[2] * c2[:, :1]) * eo2_ref[...])
    tc_ref[...] = c0 + c1 + c2


def _run_combine(w3, eos, confs, S):
    def tok_spec(width):
        return pl.BlockSpec((_TB, width), lambda i: (i, 0))

    in_specs = [pl.BlockSpec(memory_space=pltpu.SMEM)]
    in_specs += [tok_spec(_H)] * 3 + [tok_spec(_LANE)] * 3
    out_shape = [
        jax.ShapeDtypeStruct((S, _H), jnp.float32),
        jax.ShapeDtypeStruct((S, _LANE), jnp.float32),
    ]
    out_specs = [tok_spec(_H), tok_spec(_LANE)]
    return pl.pallas_call(
        _combine_body,
        grid=(S // _TB,),
        in_specs=in_specs,
        out_specs=out_specs,
        out_shape=out_shape,
    )(w3, *eos, *confs)


def kernel(hidden_states, router_params, expert_params):
    B, S, H = hidden_states.shape
    outs, specs, urgs, tcs = [], [], [], []

    def _mk_branch(p):
        n = len(p['Ws'])
        return lambda xb_: _run_expert(p, n, xb_, S)

    branches = [_mk_branch(p) for p in expert_params]
    for b in range(B):
        x = hidden_states[b]
        spec_o, urg_o, w_o, idx_o = _run_router(x, router_params, S)
        xb = x.astype(jnp.bfloat16)
        picked = [jax.lax.switch(idx_o[0, j], branches, xb)
                  for j in range(3)]
        eos = [pe[0] for pe in picked]
        confs = [pe[1] for pe in picked]
        out2d, tc = _run_combine(w_o[0, :3], eos, confs, S)
        outs.append(out2d)
        specs.append(spec_o[0, :_NSPEC])
        urgs.append(urg_o[0, :1])
        tcs.append(tc[:, :1])
    output = jnp.stack(outs).reshape(B, S, H)
    spec_probs = jnp.stack(specs)
    urgency = jnp.stack(urgs)
    total_conf = jnp.concatenate(tcs, axis=0)
    return output, spec_probs, urgency, total_conf
